# Initial kernel scaffold; baseline (speedup 1.0000x reference)
import jax
import jax.numpy as jnp
from jax.experimental import pallas as pl


def kernel(x, edge_index, W1, b1, W2, b2, gamma, beta):
    raise NotImplementedError("write your pallas kernel here")


# speedup vs baseline: 1.4475x; 1.4475x over previous
import functools

import jax
import jax.numpy as jnp
from jax import lax
from jax.experimental import pallas as pl
from jax.experimental.pallas import tpu as pltpu
from jax.experimental.pallas import tpu_sc as plsc

N_NODES = 10000
N_EDGES = 640000
D_IN = 116
HIDDEN = 256

NC = 2
NS = 16
NP = 2
CH = 128
K = 320
E_PAD = NS * K * CH
RANGE = 2560
NPC = 2688
RPS = NPC // NS
DUMP = NPC - 1
DP = 128


def _sc_aggregate(init, srcp, dstp, x):
    mesh = plsc.VectorSubcoreMesh(core_axis_name="c", subcore_axis_name="s")

    @functools.partial(
        pl.kernel,
        out_type=jax.ShapeDtypeStruct((NC, NP, NPC, DP), jnp.float32),
        mesh=mesh,
        scratch_types=[
            pltpu.VMEM((1, K, CH), jnp.int32),
            pltpu.VMEM((1, 1, 1, K, CH), jnp.int32),
            pltpu.VMEM((CH, DP), jnp.float32),
            pltpu.SemaphoreType.DMA,
            pltpu.VMEM_SHARED((NPC, DP), jnp.float32),
        ],
        compiler_params=pltpu.CompilerParams(use_tc_tiling_on_sc=False),
    )
    def agg_kernel(init_hbm, src_hbm, dst_hbm, x_hbm, out_hbm,
                   sidx, didx, rows, sem, acc):
        c = lax.axis_index("c")
        s = lax.axis_index("s")
        row0 = s * RPS
        pltpu.sync_copy(src_hbm.at[pl.ds(s, 1)], sidx)
        for p in range(NP):
            pltpu.sync_copy(init_hbm.at[c, p, pl.ds(row0, RPS)],
                            acc.at[pl.ds(row0, RPS)])
            pltpu.sync_copy(
                dst_hbm.at[pl.ds(c, 1), pl.ds(p, 1), pl.ds(s, 1)], didx)
            plsc.subcore_barrier()

            def body(j, carry):
                pltpu.async_copy(x_hbm.at[sidx.at[0, j]], rows, sem).wait()
                pltpu.sync_copy(rows, acc.at[didx.at[0, 0, 0, j]], add=True)
                return carry

            lax.fori_loop(0, K, body, 0)
            plsc.subcore_barrier()
            pltpu.sync_copy(acc.at[pl.ds(row0, RPS)],
                            out_hbm.at[c, p, pl.ds(row0, RPS)])
            plsc.subcore_barrier()

    return agg_kernel(init, srcp, dstp, x)


def _mlp_body(p_ref, w1_ref, b1_ref, w2_ref, b2_ref, g_ref, bt_ref, o_ref):
    h = jnp.concatenate(
        [p_ref[0, 0, :RANGE, :], p_ref[0, 1, :RANGE, :],
         p_ref[1, 0, :RANGE, :], p_ref[1, 1, :N_NODES - 3 * RANGE, :]],
        axis=0)
    h1 = jnp.dot(h, w1_ref[:], preferred_element_type=jnp.float32) + b1_ref[:]
    h1 = jnp.maximum(h1, 0.0)
    h2 = jnp.dot(h1, w2_ref[:], preferred_element_type=jnp.float32) + b2_ref[:]
    h2 = jnp.maximum(h2, 0.0)
    mean = jnp.mean(h2, axis=0, keepdims=True)
    cen = h2 - mean
    var = jnp.mean(cen * cen, axis=0, keepdims=True)
    o = cen * lax.rsqrt(var + 1e-5) * g_ref[:] + bt_ref[:]
    o_ref[:] = jnp.maximum(o, 0.0)


def kernel(x, edge_index, W1, b1, W2, b2, gamma, beta):
    src = edge_index[0]
    dst = edge_index[1]
    pad = E_PAD - N_EDGES
    srcp = jnp.concatenate(
        [src, jnp.zeros((pad,), jnp.int32)]).reshape(NS, K, CH)
    dstf = jnp.concatenate([dst, jnp.full((pad,), -1, jnp.int32)])
    base = jnp.arange(NC * NP, dtype=jnp.int32)[:, None] * RANGE
    loc = dstf[None, :] - base
    dstp = jnp.where((loc >= 0) & (loc < RANGE), loc,
                     DUMP).astype(jnp.int32).reshape(NC, NP, NS, K, CH)
    xp = jnp.pad(x, ((0, NC * NP * RANGE - N_NODES), (0, DP - D_IN)))
    init = xp.reshape(NC, NP, RANGE, DP)
    init = jnp.pad(init, ((0, 0), (0, 0), (0, NPC - RANGE), (0, 0)))

    partials = _sc_aggregate(init, srcp, dstp, xp)

    w1p = jnp.pad(W1, ((0, DP - D_IN), (0, 0)))
    out = pl.pallas_call(
        _mlp_body,
        out_shape=jax.ShapeDtypeStruct((N_NODES, HIDDEN), jnp.float32),
    )(partials, w1p, b1.reshape(1, HIDDEN), W2, b2.reshape(1, HIDDEN),
      gamma.reshape(1, HIDDEN), beta.reshape(1, HIDDEN))
    return out


# speedup vs baseline: 2.3714x; 1.6383x over previous
import functools

import jax
import jax.numpy as jnp
from jax import lax
from jax.experimental import pallas as pl
from jax.experimental.pallas import tpu as pltpu
from jax.experimental.pallas import tpu_sc as plsc

N_NODES = 10000
N_EDGES = 640000
D_IN = 116
HIDDEN = 256

NC = 2
NS = 16
NP = 2
CH = 128
K = 160
E_PAD = NC * NS * K * CH
HALF = 5120
NPC = 5248
RPS = NPC // NS
DUMP = NPC - 1
DP = 128


def _sc_aggregate(init, srcp, dstp, x):
    mesh = plsc.VectorSubcoreMesh(core_axis_name="c", subcore_axis_name="s")

    @functools.partial(
        pl.kernel,
        out_type=jax.ShapeDtypeStruct((NC, NP, NPC, DP), jnp.float32),
        mesh=mesh,
        scratch_types=[
            pltpu.VMEM((1, 1, K, CH), jnp.int32),
            pltpu.VMEM((1, 1, 1, K, CH), jnp.int32),
            pltpu.VMEM((CH, DP), jnp.float32),
            pltpu.VMEM((CH, DP), jnp.float32),
            pltpu.SemaphoreType.DMA,
            pltpu.SemaphoreType.DMA,
            pltpu.VMEM_SHARED((NPC, DP), jnp.float32),
        ],
        compiler_params=pltpu.CompilerParams(use_tc_tiling_on_sc=False),
    )
    def agg_kernel(init_hbm, src_hbm, dst_hbm, x_hbm, out_hbm,
                   sidx, didx, rows0, rows1, sem0, sem1, acc):
        c = lax.axis_index("c")
        s = lax.axis_index("s")
        row0 = s * RPS
        pltpu.sync_copy(src_hbm.at[pl.ds(c, 1), pl.ds(s, 1)], sidx)
        for p in range(NP):
            pltpu.sync_copy(init_hbm.at[c, p, pl.ds(row0, RPS)],
                            acc.at[pl.ds(row0, RPS)])
            pltpu.sync_copy(
                dst_hbm.at[pl.ds(c, 1), pl.ds(p, 1), pl.ds(s, 1)], didx)
            plsc.subcore_barrier()

            def body(i, carry):
                j0 = 2 * i
                j1 = 2 * i + 1
                cp0 = pltpu.async_copy(x_hbm.at[sidx.at[0, 0, j0]], rows0,
                                       sem0)
                cp1 = pltpu.async_copy(x_hbm.at[sidx.at[0, 0, j1]], rows1,
                                       sem1)
                cp0.wait()
                pltpu.sync_copy(rows0, acc.at[didx.at[0, 0, 0, j0]], add=True)
                cp1.wait()
                pltpu.sync_copy(rows1, acc.at[didx.at[0, 0, 0, j1]], add=True)
                return carry

            lax.fori_loop(0, K // 2, body, 0)
            plsc.subcore_barrier()
            pltpu.sync_copy(acc.at[pl.ds(row0, RPS)],
                            out_hbm.at[c, p, pl.ds(row0, RPS)])
            plsc.subcore_barrier()

    return agg_kernel(init, srcp, dstp, x)


def _mlp_body(p_ref, w1_ref, b1_ref, w2_ref, b2_ref, g_ref, bt_ref, o_ref):
    h = jnp.concatenate(
        [p_ref[0, 0, :HALF, :] + p_ref[1, 0, :HALF, :],
         p_ref[0, 1, :N_NODES - HALF, :] + p_ref[1, 1, :N_NODES - HALF, :]],
        axis=0)
    h1 = jnp.dot(h, w1_ref[:], preferred_element_type=jnp.float32) + b1_ref[:]
    h1 = jnp.maximum(h1, 0.0)
    h2 = jnp.dot(h1, w2_ref[:], preferred_element_type=jnp.float32) + b2_ref[:]
    h2 = jnp.maximum(h2, 0.0)
    mean = jnp.mean(h2, axis=0, keepdims=True)
    cen = h2 - mean
    var = jnp.mean(cen * cen, axis=0, keepdims=True)
    o = cen * lax.rsqrt(var + 1e-5) * g_ref[:] + bt_ref[:]
    o_ref[:] = jnp.maximum(o, 0.0)


def kernel(x, edge_index, W1, b1, W2, b2, gamma, beta):
    src = edge_index[0]
    dst = edge_index[1]
    pad = E_PAD - N_EDGES
    srcp = jnp.concatenate(
        [src, jnp.zeros((pad,), jnp.int32)]).reshape(NC, NS, K, CH)
    dstf = jnp.concatenate([dst, jnp.full((pad,), -1, jnp.int32)])
    base = jnp.arange(NP, dtype=jnp.int32)[:, None] * HALF
    loc = dstf[None, :] - base
    dstp = jnp.where((loc >= 0) & (loc < HALF), loc, DUMP).astype(jnp.int32)
    dstp = dstp.reshape(NP, NC, NS, K, CH).transpose(1, 0, 2, 3, 4)
    xp = jnp.pad(x, ((0, NP * HALF - N_NODES), (0, DP - D_IN)))
    init0 = jnp.pad(xp.reshape(NP, HALF, DP),
                    ((0, 0), (0, NPC - HALF), (0, 0)))
    init = jnp.stack([init0, jnp.zeros_like(init0)])

    partials = _sc_aggregate(init, srcp, dstp, xp)

    w1p = jnp.pad(W1, ((0, DP - D_IN), (0, 0)))
    out = pl.pallas_call(
        _mlp_body,
        out_shape=jax.ShapeDtypeStruct((N_NODES, HIDDEN), jnp.float32),
    )(partials, w1p, b1.reshape(1, HIDDEN), W2, b2.reshape(1, HIDDEN),
      gamma.reshape(1, HIDDEN), beta.reshape(1, HIDDEN))
    return out


# speedup vs baseline: 2.5992x; 1.0961x over previous
import functools

import jax
import jax.numpy as jnp
from jax import lax
from jax.experimental import pallas as pl
from jax.experimental.pallas import tpu as pltpu
from jax.experimental.pallas import tpu_sc as plsc

N_NODES = 10000
N_EDGES = 640000
D_IN = 116
HIDDEN = 256

NC = 2
NS = 16
NP = 2
CH = 128
K = 160
E_PAD = NC * NS * K * CH
HALF = 5120
NPC = 5248
RPS = NPC // NS
DUMP = NPC - 1
DP = 128


def _sc_aggregate(init, srcp, dstp, x):
    mesh = plsc.VectorSubcoreMesh(core_axis_name="c", subcore_axis_name="s")

    @functools.partial(
        pl.kernel,
        out_type=jax.ShapeDtypeStruct((NC, NP, NPC, DP), jnp.float32),
        mesh=mesh,
        scratch_types=[
            pltpu.VMEM((1, 1, K, CH), jnp.int32),
            pltpu.VMEM((1, 1, 1, K, CH), jnp.int32),
            pltpu.VMEM((CH, DP), jnp.float32),
            pltpu.VMEM((CH, DP), jnp.float32),
            pltpu.SemaphoreType.DMA,
            pltpu.SemaphoreType.DMA,
            pltpu.VMEM_SHARED((NPC, DP), jnp.float32),
        ],
        compiler_params=pltpu.CompilerParams(use_tc_tiling_on_sc=False),
    )
    def agg_kernel(init_hbm, src_hbm, dst_hbm, x_hbm, out_hbm,
                   sidx, didx, rows0, rows1, sem0, sem1, acc):
        c = lax.axis_index("c")
        s = lax.axis_index("s")
        row0 = s * RPS
        pltpu.sync_copy(src_hbm.at[pl.ds(c, 1), pl.ds(s, 1)], sidx)
        for p in range(NP):
            pltpu.sync_copy(init_hbm.at[c, p, pl.ds(row0, RPS)],
                            acc.at[pl.ds(row0, RPS)])
            pltpu.sync_copy(
                dst_hbm.at[pl.ds(c, 1), pl.ds(p, 1), pl.ds(s, 1)], didx)
            plsc.subcore_barrier()

            def body(i, carry):
                j0 = 2 * i
                j1 = 2 * i + 1
                cp0 = pltpu.async_copy(x_hbm.at[sidx.at[0, 0, j0]], rows0,
                                       sem0)
                cp1 = pltpu.async_copy(x_hbm.at[sidx.at[0, 0, j1]], rows1,
                                       sem1)
                cp0.wait()
                pltpu.sync_copy(rows0, acc.at[didx.at[0, 0, 0, j0]], add=True)
                cp1.wait()
                pltpu.sync_copy(rows1, acc.at[didx.at[0, 0, 0, j1]], add=True)
                return carry

            lax.fori_loop(0, K // 2, body, 0)
            plsc.subcore_barrier()
            pltpu.sync_copy(acc.at[pl.ds(row0, RPS)],
                            out_hbm.at[c, p, pl.ds(row0, RPS)])
            plsc.subcore_barrier()

    return agg_kernel(init, srcp, dstp, x)


def _mlp_body(p_ref, w1_ref, b1_ref, w2_ref, b2_ref, g_ref, bt_ref, o_ref):
    h = jnp.concatenate(
        [p_ref[0, 0, :HALF, :] + p_ref[1, 0, :HALF, :],
         p_ref[0, 1, :N_NODES - HALF, :] + p_ref[1, 1, :N_NODES - HALF, :]],
        axis=0)
    h1 = jnp.dot(h, w1_ref[:], preferred_element_type=jnp.float32) + b1_ref[:]
    h1 = jnp.maximum(h1, 0.0)
    h2 = jnp.dot(h1, w2_ref[:], preferred_element_type=jnp.float32) + b2_ref[:]
    h2 = jnp.maximum(h2, 0.0)
    mean = jnp.mean(h2, axis=0, keepdims=True)
    cen = h2 - mean
    var = jnp.mean(cen * cen, axis=0, keepdims=True)
    o = cen * lax.rsqrt(var + 1e-5) * g_ref[:] + bt_ref[:]
    o_ref[:] = jnp.maximum(o, 0.0)


def kernel(x, edge_index, W1, b1, W2, b2, gamma, beta):
    src = edge_index[0]
    dst = edge_index[1]
    pad = E_PAD - N_EDGES
    srcp = jnp.concatenate(
        [src, jnp.zeros((pad,), jnp.int32)]).reshape(NC, NS, K, CH)
    dstf = jnp.concatenate([dst, jnp.full((pad,), -1, jnp.int32)])
    spread = HALF + (jnp.arange(E_PAD, dtype=jnp.int32) & (NPC - HALF - 1))
    base = jnp.arange(NP, dtype=jnp.int32)[:, None] * HALF
    loc = dstf[None, :] - base
    dstp = jnp.where((loc >= 0) & (loc < HALF), loc,
                     spread[None, :]).astype(jnp.int32)
    dstp = dstp.reshape(NP, NC, NS, K, CH).transpose(1, 0, 2, 3, 4)
    xp = jnp.pad(x, ((0, NP * HALF - N_NODES), (0, DP - D_IN)))
    init0 = jnp.pad(xp.reshape(NP, HALF, DP),
                    ((0, 0), (0, NPC - HALF), (0, 0)))
    init = jnp.stack([init0, jnp.zeros_like(init0)])

    partials = _sc_aggregate(init, srcp, dstp, xp)

    w1p = jnp.pad(W1, ((0, DP - D_IN), (0, 0)))
    out = pl.pallas_call(
        _mlp_body,
        out_shape=jax.ShapeDtypeStruct((N_NODES, HIDDEN), jnp.float32),
    )(partials, w1p, b1.reshape(1, HIDDEN), W2, b2.reshape(1, HIDDEN),
      gamma.reshape(1, HIDDEN), beta.reshape(1, HIDDEN))
    return out


# speedup vs baseline: 4.3467x; 1.6723x over previous
import functools

import jax
import jax.numpy as jnp
from jax import lax
from jax.experimental import pallas as pl
from jax.experimental.pallas import tpu as pltpu
from jax.experimental.pallas import tpu_sc as plsc

N_NODES = 10000
N_EDGES = 640000
D_IN = 116
HIDDEN = 256

NC = 2
NS = 16
CH = 128
K = 160
KS = 40
NST = K // KS
E_PAD = NC * NS * K * CH
NPC = 10496
RPS = NPC // NS
DUMP0 = 10240
DP = 128


def _sc_aggregate(init, srcp, dstp, x):
    mesh = plsc.VectorSubcoreMesh(core_axis_name="c", subcore_axis_name="s")

    @functools.partial(
        pl.kernel,
        out_type=jax.ShapeDtypeStruct((NC, NPC, DP), jnp.float32),
        mesh=mesh,
        scratch_types=[
            pltpu.VMEM((1, 1, KS, CH), jnp.int32),
            pltpu.VMEM((1, 1, KS, CH), jnp.int32),
            pltpu.VMEM((CH, DP), jnp.float32),
            pltpu.VMEM((CH, DP), jnp.float32),
            pltpu.SemaphoreType.DMA,
            pltpu.SemaphoreType.DMA,
            pltpu.VMEM_SHARED((NPC, DP), jnp.float32),
        ],
        compiler_params=pltpu.CompilerParams(use_tc_tiling_on_sc=False),
    )
    def agg_kernel(init_hbm, src_hbm, dst_hbm, x_hbm, out_hbm,
                   sidx, didx, rows0, rows1, sem0, sem1, acc):
        c = lax.axis_index("c")
        s = lax.axis_index("s")
        row0 = s * RPS
        pltpu.sync_copy(init_hbm.at[c, pl.ds(row0, RPS)],
                        acc.at[pl.ds(row0, RPS)])
        plsc.subcore_barrier()
        for st in range(NST):
            pltpu.sync_copy(
                src_hbm.at[pl.ds(c, 1), pl.ds(s, 1), pl.ds(st * KS, KS)],
                sidx)
            pltpu.sync_copy(
                dst_hbm.at[pl.ds(c, 1), pl.ds(s, 1), pl.ds(st * KS, KS)],
                didx)

            def body(i, carry):
                j0 = 2 * i
                j1 = 2 * i + 1
                cp0 = pltpu.async_copy(x_hbm.at[sidx.at[0, 0, j0]], rows0,
                                       sem0)
                cp1 = pltpu.async_copy(x_hbm.at[sidx.at[0, 0, j1]], rows1,
                                       sem1)
                cp0.wait()
                pltpu.sync_copy(rows0, acc.at[didx.at[0, 0, j0]], add=True)
                cp1.wait()
                pltpu.sync_copy(rows1, acc.at[didx.at[0, 0, j1]], add=True)
                return carry

            lax.fori_loop(0, KS // 2, body, 0)
        plsc.subcore_barrier()
        pltpu.sync_copy(acc.at[pl.ds(row0, RPS)],
                        out_hbm.at[c, pl.ds(row0, RPS)])

    return agg_kernel(init, srcp, dstp, x)


def _mlp_body(p_ref, w1_ref, b1_ref, w2_ref, b2_ref, g_ref, bt_ref, o_ref):
    h = p_ref[0, :N_NODES, :] + p_ref[1, :N_NODES, :]
    h1 = jnp.dot(h, w1_ref[:], preferred_element_type=jnp.float32) + b1_ref[:]
    h1 = jnp.maximum(h1, 0.0)
    h2 = jnp.dot(h1, w2_ref[:], preferred_element_type=jnp.float32) + b2_ref[:]
    h2 = jnp.maximum(h2, 0.0)
    mean = jnp.mean(h2, axis=0, keepdims=True)
    cen = h2 - mean
    var = jnp.mean(cen * cen, axis=0, keepdims=True)
    o = cen * lax.rsqrt(var + 1e-5) * g_ref[:] + bt_ref[:]
    o_ref[:] = jnp.maximum(o, 0.0)


def kernel(x, edge_index, W1, b1, W2, b2, gamma, beta):
    src = edge_index[0]
    dst = edge_index[1]
    pad = E_PAD - N_EDGES
    srcp = jnp.concatenate(
        [src, jnp.zeros((pad,), jnp.int32)]).reshape(NC, NS, K, CH)
    dstf = jnp.concatenate([dst, jnp.full((pad,), -1, jnp.int32)])
    spread = DUMP0 + (jnp.arange(E_PAD, dtype=jnp.int32) & 255)
    dstp = jnp.where((dstf >= 0) & (dstf < N_NODES), dstf,
                     spread).astype(jnp.int32).reshape(NC, NS, K, CH)
    xp = jnp.pad(x, ((0, DUMP0 - N_NODES), (0, DP - D_IN)))
    init0 = jnp.pad(xp, ((0, NPC - DUMP0), (0, 0)))
    init = jnp.stack([init0, jnp.zeros_like(init0)])

    partials = _sc_aggregate(init, srcp, dstp, xp)

    w1p = jnp.pad(W1, ((0, DP - D_IN), (0, 0)))
    out = pl.pallas_call(
        _mlp_body,
        out_shape=jax.ShapeDtypeStruct((N_NODES, HIDDEN), jnp.float32),
    )(partials, w1p, b1.reshape(1, HIDDEN), W2, b2.reshape(1, HIDDEN),
      gamma.reshape(1, HIDDEN), beta.reshape(1, HIDDEN))
    return out
